# Initial kernel scaffold; baseline (speedup 1.0000x reference)
import jax
import jax.numpy as jnp
from jax.experimental import pallas as pl


def kernel(xyz, points, mlp_params):
    raise NotImplementedError("write your pallas kernel here")


# speedup vs baseline: 14.2656x; 14.2656x over previous
import functools

import jax
import jax.numpy as jnp
import numpy as np
from jax import lax
from jax.experimental import pallas as pl
from jax.experimental.pallas import tpu as pltpu
from jax.experimental.pallas import tpu_sc as plsc

B = 8
N = 4096
S = 1024
K = 32
RSQ = np.float32(0.2 ** 2)
BIG = np.float32(1e10)
F32 = jnp.float32


def _fps_body(xt_ref, f0_ref, out_ref):
    x = xt_ref[0]
    y = xt_ref[1]
    z = xt_ref[2]
    lane_n = lax.broadcasted_iota(jnp.int32, (B, N), 1)
    lane_s = lax.broadcasted_iota(jnp.int32, (B, S), 1)

    def body(i, carry):
        dist, far, ox, oy, oz = carry
        onehot = lane_n == far
        cx = jnp.sum(jnp.where(onehot, x, 0.0), axis=1, keepdims=True)
        cy = jnp.sum(jnp.where(onehot, y, 0.0), axis=1, keepdims=True)
        cz = jnp.sum(jnp.where(onehot, z, 0.0), axis=1, keepdims=True)
        col = lane_s == i
        ox = jnp.where(col, cx, ox)
        oy = jnp.where(col, cy, oy)
        oz = jnp.where(col, cz, oz)
        dx = x - cx
        dy = y - cy
        dz = z - cz
        d = (dx * dx + dy * dy) + dz * dz
        dist = jnp.where(d < dist, d, dist)
        m = jnp.max(dist, axis=1, keepdims=True)
        far = jnp.min(jnp.where(dist == m, lane_n, jnp.int32(N)),
                      axis=1, keepdims=True)
        return dist, far, ox, oy, oz

    dist0 = jnp.full((B, N), BIG, dtype=F32)
    far0 = f0_ref[:, 0:1]
    o0 = jnp.zeros((B, S), dtype=F32)
    _, _, ox, oy, oz = lax.fori_loop(0, S, body, (dist0, far0, o0, o0, o0))
    out_ref[0] = ox
    out_ref[1] = oy
    out_ref[2] = oz


def _fps(xt, far0_2d):
    return pl.pallas_call(
        _fps_body,
        out_shape=jax.ShapeDtypeStruct((3, B, S), F32),
    )(xt, far0_2d)


def _tables_body(xyz_ref, pts_ref, nq_ref, w1xt_ref, w1pt_ref, b1_ref,
                 t1_ref, u1_ref):
    xb = xyz_ref[0]
    pb = pts_ref[0]
    nb = nq_ref[0]
    w1xt = w1xt_ref[...]
    t = jnp.dot(xb, w1xt, preferred_element_type=F32)
    t = t + jnp.dot(pb, w1pt_ref[...], preferred_element_type=F32)
    t1_ref[0] = jnp.concatenate([t + b1_ref[...], jnp.zeros((N, 64), F32)],
                                axis=1)
    u1_ref[0] = jnp.dot(nb, w1xt, preferred_element_type=F32)


def _tables(xyz, points, new_xyz, w1xt, w1pt, b1row):
    return pl.pallas_call(
        _tables_body,
        grid=(B,),
        in_specs=[
            pl.BlockSpec((1, N, 3), lambda b: (b, 0, 0)),
            pl.BlockSpec((1, N, 64), lambda b: (b, 0, 0)),
            pl.BlockSpec((1, S, 3), lambda b: (b, 0, 0)),
            pl.BlockSpec((3, 64), lambda b: (0, 0)),
            pl.BlockSpec((64, 64), lambda b: (0, 0)),
            pl.BlockSpec((1, 64), lambda b: (0, 0)),
        ],
        out_specs=[
            pl.BlockSpec((1, N, 128), lambda b: (b, 0, 0)),
            pl.BlockSpec((1, S, 64), lambda b: (b, 0, 0)),
        ],
        out_shape=[
            jax.ShapeDtypeStruct((B, N, 128), F32),
            jax.ShapeDtypeStruct((B, S, 64), F32),
        ],
    )(xyz, points, new_xyz, w1xt, w1pt, b1row)


QB = 256


def _ballq_body(nq_ref, xt_ref, gidx_ref):
    q = nq_ref[0]
    xt = xt_ref[0]
    mm = lax.dot_general(q, xt, (((1,), (0,)), ((), ())),
                         preferred_element_type=F32)
    dist = -2.0 * mm
    dist = dist + jnp.sum(q * q, axis=1, keepdims=True)
    dist = dist + jnp.sum(xt * xt, axis=0, keepdims=True)
    mask = dist > RSQ
    cnt = jnp.sum(mask.astype(jnp.int32), axis=1, keepdims=True)
    d = jnp.where(mask, BIG, dist)
    lane_n = lax.broadcasted_iota(jnp.int32, (QB, N), 1)
    sels = []
    for _ in range(K):
        m = jnp.min(d, axis=1, keepdims=True)
        idx = jnp.min(jnp.where(d == m, lane_n, jnp.int32(N)),
                      axis=1, keepdims=True)
        sels.append(idx)
        d = jnp.where(lane_n == idx, np.float32(np.inf), d)
    sel = jnp.concatenate(sels, axis=1)
    sel = jnp.where(cnt < K, sel[:, 0:1], sel)
    gidx_ref[0] = sel + pl.program_id(0) * N


def _ballq(new_xyz, xT):
    return pl.pallas_call(
        _ballq_body,
        grid=(B, S // QB),
        in_specs=[
            pl.BlockSpec((1, QB, 3), lambda b, s: (b, s, 0)),
            pl.BlockSpec((1, 3, N), lambda b, s: (b, 0, 0)),
        ],
        out_specs=pl.BlockSpec((1, QB, K), lambda b, s: (b, s, 0)),
        out_shape=jax.ShapeDtypeStruct((B, S, K), jnp.int32),
    )(new_xyz, xT)


def _sc_gather(table, idx_flat):
    rows = B * S * K
    info = plsc.get_sparse_core_info()
    nw = info.num_cores * info.num_subcores
    b_per_w = rows // nw
    chunk = 512
    nchunk = b_per_w // chunk
    mesh = plsc.VectorSubcoreMesh(core_axis_name="c", subcore_axis_name="s")

    @functools.partial(
        pl.kernel, mesh=mesh,
        out_type=jax.ShapeDtypeStruct((rows, 128), F32),
        scratch_types=[
            pltpu.VMEM((chunk,), jnp.int32),
            pltpu.VMEM((chunk, 128), F32),
            pltpu.SemaphoreType.DMA,
        ],
    )
    def gk(table_hbm, idx_hbm, out_hbm, idx_v, rows_v, sem):
        wid = lax.axis_index("s") * info.num_cores + lax.axis_index("c")
        base = wid * b_per_w

        @pl.loop(0, nchunk)
        def _(c):
            o = base + c * chunk
            pltpu.sync_copy(idx_hbm.at[pl.ds(o, chunk)], idx_v)
            pltpu.async_copy(table_hbm.at[idx_v], rows_v, sem).wait()
            pltpu.sync_copy(rows_v, out_hbm.at[pl.ds(o, chunk)])

    return gk(table, idx_flat)


QM = 128


def _z1_body(g_ref, u_ref, z1_ref, s_ref, q_ref):
    zz = g_ref[0][:, :, :64] - u_ref[0][:, None, :]
    z1_ref[0] = zz
    flat = zz.reshape(QM * K, 64)
    ps = jnp.sum(flat, axis=0, keepdims=True)
    pq = jnp.sum(flat * flat, axis=0, keepdims=True)
    first = (pl.program_id(0) == 0) & (pl.program_id(1) == 0)

    @pl.when(first)
    def _():
        s_ref[...] = jnp.broadcast_to(ps, (8, 64))
        q_ref[...] = jnp.broadcast_to(pq, (8, 64))

    @pl.when(jnp.logical_not(first))
    def _():
        s_ref[...] += jnp.broadcast_to(ps, (8, 64))
        q_ref[...] += jnp.broadcast_to(pq, (8, 64))


def _z1_pass(g4, u1):
    return pl.pallas_call(
        _z1_body,
        grid=(B, S // QM),
        in_specs=[
            pl.BlockSpec((1, QM, K, 128), lambda b, s: (b, s, 0, 0)),
            pl.BlockSpec((1, QM, 64), lambda b, s: (b, s, 0)),
        ],
        out_specs=[
            pl.BlockSpec((1, QM, K, 64), lambda b, s: (b, s, 0, 0)),
            pl.BlockSpec((8, 64), lambda b, s: (0, 0)),
            pl.BlockSpec((8, 64), lambda b, s: (0, 0)),
        ],
        out_shape=[
            jax.ShapeDtypeStruct((B, S, K, 64), F32),
            jax.ShapeDtypeStruct((8, 64), F32),
            jax.ShapeDtypeStruct((8, 64), F32),
        ],
    )(g4, u1)


def _mid_body(z_ref, a_ref, c_ref, wt_ref, b_ref, zo_ref, s_ref, q_ref, *,
              cout):
    x = z_ref[0].reshape(QM * K, 64)
    x = jnp.maximum(a_ref[...] * x + c_ref[...], 0.0)
    zn = jnp.dot(x, wt_ref[...], preferred_element_type=F32) + b_ref[...]
    zo_ref[0] = zn.reshape(QM, K, cout)
    ps = jnp.sum(zn, axis=0, keepdims=True)
    pq = jnp.sum(zn * zn, axis=0, keepdims=True)
    first = (pl.program_id(0) == 0) & (pl.program_id(1) == 0)

    @pl.when(first)
    def _():
        s_ref[...] = jnp.broadcast_to(ps, (8, cout))
        q_ref[...] = jnp.broadcast_to(pq, (8, cout))

    @pl.when(jnp.logical_not(first))
    def _():
        s_ref[...] += jnp.broadcast_to(ps, (8, cout))
        q_ref[...] += jnp.broadcast_to(pq, (8, cout))


def _mid_pass(z, a, c, wt, brow, cout):
    return pl.pallas_call(
        functools.partial(_mid_body, cout=cout),
        grid=(B, S // QM),
        in_specs=[
            pl.BlockSpec((1, QM, K, 64), lambda b, s: (b, s, 0, 0)),
            pl.BlockSpec((1, 64), lambda b, s: (0, 0)),
            pl.BlockSpec((1, 64), lambda b, s: (0, 0)),
            pl.BlockSpec((64, cout), lambda b, s: (0, 0)),
            pl.BlockSpec((1, cout), lambda b, s: (0, 0)),
        ],
        out_specs=[
            pl.BlockSpec((1, QM, K, cout), lambda b, s: (b, s, 0, 0)),
            pl.BlockSpec((8, cout), lambda b, s: (0, 0)),
            pl.BlockSpec((8, cout), lambda b, s: (0, 0)),
        ],
        out_shape=[
            jax.ShapeDtypeStruct((B, S, K, cout), F32),
            jax.ShapeDtypeStruct((8, cout), F32),
            jax.ShapeDtypeStruct((8, cout), F32),
        ],
    )(z, a, c, wt, brow)


def _l3_body(z_ref, a_ref, c_ref, wt_ref, b_ref,
             zmax_ref, zmin_ref, s_ref, q_ref):
    x = z_ref[0].reshape(QM * K, 64)
    x = jnp.maximum(a_ref[...] * x + c_ref[...], 0.0)
    zn = jnp.dot(x, wt_ref[...], preferred_element_type=F32) + b_ref[...]
    ps = jnp.sum(zn, axis=0, keepdims=True)
    pq = jnp.sum(zn * zn, axis=0, keepdims=True)
    r = zn.reshape(QM, K, 128)
    zmax_ref[0] = jnp.max(r, axis=1)
    zmin_ref[0] = jnp.min(r, axis=1)
    first = (pl.program_id(0) == 0) & (pl.program_id(1) == 0)

    @pl.when(first)
    def _():
        s_ref[...] = jnp.broadcast_to(ps, (8, 128))
        q_ref[...] = jnp.broadcast_to(pq, (8, 128))

    @pl.when(jnp.logical_not(first))
    def _():
        s_ref[...] += jnp.broadcast_to(ps, (8, 128))
        q_ref[...] += jnp.broadcast_to(pq, (8, 128))


def _l3_pass(z, a, c, wt, brow):
    return pl.pallas_call(
        _l3_body,
        grid=(B, S // QM),
        in_specs=[
            pl.BlockSpec((1, QM, K, 64), lambda b, s: (b, s, 0, 0)),
            pl.BlockSpec((1, 64), lambda b, s: (0, 0)),
            pl.BlockSpec((1, 64), lambda b, s: (0, 0)),
            pl.BlockSpec((64, 128), lambda b, s: (0, 0)),
            pl.BlockSpec((1, 128), lambda b, s: (0, 0)),
        ],
        out_specs=[
            pl.BlockSpec((1, QM, 128), lambda b, s: (b, s, 0)),
            pl.BlockSpec((1, QM, 128), lambda b, s: (b, s, 0)),
            pl.BlockSpec((8, 128), lambda b, s: (0, 0)),
            pl.BlockSpec((8, 128), lambda b, s: (0, 0)),
        ],
        out_shape=[
            jax.ShapeDtypeStruct((B, S, 128), F32),
            jax.ShapeDtypeStruct((B, S, 128), F32),
            jax.ShapeDtypeStruct((8, 128), F32),
            jax.ShapeDtypeStruct((8, 128), F32),
        ],
    )(z, a, c, wt, brow)


def _final_body(zmax_ref, zmin_ref, a_ref, c_ref, out_ref):
    a = a_ref[...]
    zm = jnp.where(a >= 0.0, zmax_ref[0], zmin_ref[0])
    out_ref[0] = jnp.maximum(a * zm + c_ref[...], 0.0)


def _final_pass(zmax, zmin, a, c):
    return pl.pallas_call(
        _final_body,
        grid=(B,),
        in_specs=[
            pl.BlockSpec((1, S, 128), lambda b: (b, 0, 0)),
            pl.BlockSpec((1, S, 128), lambda b: (b, 0, 0)),
            pl.BlockSpec((1, 128), lambda b: (0, 0)),
            pl.BlockSpec((1, 128), lambda b: (0, 0)),
        ],
        out_specs=pl.BlockSpec((1, S, 128), lambda b: (b, 0, 0)),
        out_shape=jax.ShapeDtypeStruct((B, S, 128), F32),
    )(zmax, zmin, a, c)


def _bn_fold(ssum, sqsum, gamma, beta):
    t = np.float32(B * S * K)
    mean = ssum / t
    var = sqsum / t - mean * mean
    a = gamma / jnp.sqrt(var + 1e-5)
    c = beta - mean * a
    return a[None, :], c[None, :]


def kernel(xyz, points, mlp_params):
    (w1, b1, g1, be1), (w2, b2, g2, be2), (w3, b3, g3, be3) = mlp_params
    xyz = xyz.astype(F32)
    points = points.astype(F32)

    far0 = jax.random.randint(jax.random.key(42), (B,), 0, N).astype(jnp.int32)
    far0_2d = jnp.broadcast_to(far0[:, None], (B, 128))

    xt = jnp.transpose(xyz, (2, 0, 1))
    nx = _fps(xt, far0_2d)
    new_xyz = jnp.transpose(nx, (1, 2, 0))

    w1xt = jnp.transpose(w1[:, :3])
    w1pt = jnp.transpose(w1[:, 3:])
    t1, u1 = _tables(xyz, points, new_xyz, w1xt, w1pt, b1[None, :])

    xT = jnp.transpose(xyz, (0, 2, 1))
    gidx = _ballq(new_xyz, xT)

    g = _sc_gather(t1.reshape(B * N, 128), gidx.reshape(B * S * K))
    g4 = g.reshape(B, S, K, 128)

    z1, s1, q1 = _z1_pass(g4, u1)
    a1, c1 = _bn_fold(s1[0], q1[0], g1, be1)

    z2, s2, q2 = _mid_pass(z1, a1, c1, jnp.transpose(w2), b2[None, :], 64)
    a2, c2 = _bn_fold(s2[0], q2[0], g2, be2)

    zmax, zmin, s3, q3 = _l3_pass(z2, a2, c2, jnp.transpose(w3), b3[None, :])
    a3, c3 = _bn_fold(s3[0], q3[0], g3, be3)

    out = _final_pass(zmax, zmin, a3, c3)
    return (new_xyz, out)
